# Initial kernel scaffold; baseline (speedup 1.0000x reference)
import jax
import jax.numpy as jnp
from jax.experimental import pallas as pl


def kernel(H, LiMEs):
    raise NotImplementedError("write your pallas kernel here")


# speedup vs baseline: 12.4205x; 12.4205x over previous
import jax
import jax.numpy as jnp
from jax.experimental import pallas as pl
from jax.experimental.pallas import tpu as pltpu

_E = 64
_K = 8
_TEMP = 1.0
_EPS = 1e-6
_TILE = 512


def _mix_kernel(hall_ref, limes_ref, pmix_ref, idx_ref, scale_ref):
    i = pl.program_id(0)

    @pl.when(i == 0)
    def _():
        scale_ref[0, 0] = jnp.maximum(jnp.max(jnp.abs(hall_ref[...])), _EPS)

    inv = (1.0 / scale_ref[0, 0]) * (1.0 / max(_TEMP, _EPS))
    logits = hall_ref[pl.ds(i * _TILE, _TILE), :] * inv
    m = jnp.max(logits, axis=-1, keepdims=True)
    p = jnp.exp(logits - m)
    z = jnp.sum(p, axis=-1, keepdims=True)
    probs = p / z
    iota = jax.lax.broadcasted_iota(jnp.int32, probs.shape, 1)
    work = probs
    mask = jnp.zeros(probs.shape, jnp.bool_)
    cols = []
    for _ in range(_K):
        mx = jnp.max(work, axis=-1, keepdims=True)
        is_max = work == mx
        first = jnp.min(jnp.where(is_max, iota, _E), axis=-1, keepdims=True)
        sel = iota == first
        cols.append(first)
        mask = jnp.logical_or(mask, sel)
        work = jnp.where(sel, -1.0, work)
    idx_ref[...] = jnp.concatenate(cols, axis=1)
    wm = jnp.where(mask, probs, 0.0)
    wsum = jnp.sum(wm, axis=-1, keepdims=True)
    w = wm / jnp.clip(wsum, 1e-9, None)
    pmix_ref[...] = jnp.dot(w, limes_ref[...], preferred_element_type=jnp.float32)


@jax.jit
def kernel(H, LiMEs):
    B, T, D = H.shape
    N = B * T
    Hs = H.reshape(N, D)[:, :_E]

    grid = (N // _TILE,)
    pmix, idx = pl.pallas_call(
        _mix_kernel,
        grid=grid,
        in_specs=[
            pl.BlockSpec((N, _E), lambda i: (0, 0)),
            pl.BlockSpec((_E, D), lambda i: (0, 0)),
        ],
        out_specs=[
            pl.BlockSpec((_TILE, D), lambda i: (i, 0)),
            pl.BlockSpec((_TILE, _K), lambda i: (i, 0)),
        ],
        out_shape=[
            jax.ShapeDtypeStruct((N, D), jnp.float32),
            jax.ShapeDtypeStruct((N, _K), jnp.int32),
        ],
        scratch_shapes=[pltpu.SMEM((1, 1), jnp.float32)],
    )(Hs, LiMEs)

    return pmix.reshape(B, T, D), idx.reshape(B, T, _K)


# speedup vs baseline: 13.7580x; 1.1077x over previous
import jax
import jax.numpy as jnp
from jax.experimental import pallas as pl
from jax.experimental.pallas import tpu as pltpu

_E = 64
_K = 8
_TEMP = 1.0
_EPS = 1e-6
_TILE = 1024


def _mix_kernel(hall_ref, limes_ref, pmix_ref, idx_ref, scale_ref):
    i = pl.program_id(0)

    @pl.when(i == 0)
    def _():
        scale_ref[0, 0] = jnp.maximum(jnp.max(jnp.abs(hall_ref[...])), _EPS)

    inv = (1.0 / scale_ref[0, 0]) * (1.0 / max(_TEMP, _EPS))
    logits = hall_ref[pl.ds(i * _TILE, _TILE), :] * inv
    m = jnp.max(logits, axis=-1, keepdims=True)
    p = jnp.exp(logits - m)
    z = jnp.sum(p, axis=-1, keepdims=True)
    probs = p / z
    iota = jax.lax.broadcasted_iota(jnp.int32, probs.shape, 1)
    work = probs
    mask = jnp.zeros(probs.shape, jnp.bool_)
    cols = []
    for _ in range(_K):
        mx = jnp.max(work, axis=-1, keepdims=True)
        is_max = work == mx
        first = jnp.min(jnp.where(is_max, iota, _E), axis=-1, keepdims=True)
        sel = iota == first
        cols.append(first)
        mask = jnp.logical_or(mask, sel)
        work = jnp.where(sel, -1.0, work)
    idx_ref[...] = jnp.concatenate(cols, axis=1)
    wm = jnp.where(mask, probs, 0.0)
    wsum = jnp.sum(wm, axis=-1, keepdims=True)
    w = wm / jnp.clip(wsum, 1e-9, None)
    pmix_ref[...] = jnp.dot(w, limes_ref[...], preferred_element_type=jnp.float32)


@jax.jit
def kernel(H, LiMEs):
    B, T, D = H.shape
    N = B * T
    Hs = H.reshape(N, D)[:, :_E]

    grid = (N // _TILE,)
    pmix, idx = pl.pallas_call(
        _mix_kernel,
        grid=grid,
        in_specs=[
            pl.BlockSpec((N, _E), lambda i: (0, 0)),
            pl.BlockSpec((_E, D), lambda i: (0, 0)),
        ],
        out_specs=[
            pl.BlockSpec((_TILE, D), lambda i: (i, 0)),
            pl.BlockSpec((_TILE, _K), lambda i: (i, 0)),
        ],
        out_shape=[
            jax.ShapeDtypeStruct((N, D), jnp.float32),
            jax.ShapeDtypeStruct((N, _K), jnp.int32),
        ],
        scratch_shapes=[pltpu.SMEM((1, 1), jnp.float32)],
    )(Hs, LiMEs)

    return pmix.reshape(B, T, D), idx.reshape(B, T, _K)


# speedup vs baseline: 17.6954x; 1.2862x over previous
import jax
import jax.numpy as jnp
from jax.experimental import pallas as pl
from jax.experimental.pallas import tpu as pltpu

_E = 64
_K = 8
_TEMP = 1.0
_EPS = 1e-6
_TILE = 1024


def _mix_kernel(hall_ref, limes_ref, pmix_ref, idx_ref, scale_ref):
    i = pl.program_id(0)

    @pl.when(i == 0)
    def _():
        scale_ref[0, 0] = jnp.maximum(jnp.max(jnp.abs(hall_ref[...])), _EPS)

    inv = (1.0 / scale_ref[0, 0]) * (1.0 / max(_TEMP, _EPS))
    logits = hall_ref[:, pl.ds(i * _TILE, _TILE)] * inv
    m = jnp.max(logits, axis=0, keepdims=True)
    p = jnp.exp(logits - m)
    iota = jax.lax.broadcasted_iota(jnp.int32, logits.shape, 0)
    work = logits
    mask = jnp.zeros(logits.shape, jnp.bool_)
    rows = []
    for _ in range(_K):
        mx = jnp.max(work, axis=0, keepdims=True)
        is_max = work == mx
        first = jnp.min(jnp.where(is_max, iota, _E), axis=0, keepdims=True)
        sel = iota == first
        rows.append(first)
        mask = jnp.logical_or(mask, sel)
        work = jnp.where(sel, -3.0e38, work)
    idx_ref[...] = jnp.concatenate(rows, axis=0)
    wm = jnp.where(mask, p, 0.0)
    wsum = jnp.sum(wm, axis=0, keepdims=True)
    w = wm / wsum
    pmix_ref[...] = jax.lax.dot_general(
        w, limes_ref[...], (((0,), (0,)), ((), ())),
        preferred_element_type=jnp.float32,
    )


@jax.jit
def kernel(H, LiMEs):
    B, T, D = H.shape
    N = B * T
    HsT = H.reshape(N, D)[:, :_E].T

    grid = (N // _TILE,)
    pmix, idx = pl.pallas_call(
        _mix_kernel,
        grid=grid,
        in_specs=[
            pl.BlockSpec((_E, N), lambda i: (0, 0)),
            pl.BlockSpec((_E, D), lambda i: (0, 0)),
        ],
        out_specs=[
            pl.BlockSpec((_TILE, D), lambda i: (i, 0)),
            pl.BlockSpec((_K, _TILE), lambda i: (0, i)),
        ],
        out_shape=[
            jax.ShapeDtypeStruct((N, D), jnp.float32),
            jax.ShapeDtypeStruct((_K, N), jnp.int32),
        ],
        scratch_shapes=[pltpu.SMEM((1, 1), jnp.float32)],
    )(HsT, LiMEs)

    return pmix.reshape(B, T, D), idx.T.reshape(B, T, _K)


# speedup vs baseline: 18.2630x; 1.0321x over previous
import jax
import jax.numpy as jnp
from jax.experimental import pallas as pl
from jax.experimental.pallas import tpu as pltpu

_E = 64
_K = 8
_TEMP = 1.0
_EPS = 1e-6
_TILE = 512


def _mix_kernel(hall_ref, limes_ref, pmix_ref, idx_ref, scale_ref):
    i = pl.program_id(0)

    @pl.when(i == 0)
    def _():
        scale_ref[0, 0] = jnp.maximum(jnp.max(jnp.abs(hall_ref[...])), _EPS)

    inv = (1.0 / scale_ref[0, 0]) * (1.0 / max(_TEMP, _EPS))
    logits = hall_ref[:, pl.ds(i * _TILE, _TILE)] * inv
    m = jnp.max(logits, axis=0, keepdims=True)
    p = jnp.exp(logits - m)
    iota = jax.lax.broadcasted_iota(jnp.int32, logits.shape, 0)
    work = logits
    mask = jnp.zeros(logits.shape, jnp.bool_)
    rows = []
    for _ in range(_K):
        mx = jnp.max(work, axis=0, keepdims=True)
        is_max = work == mx
        first = jnp.min(jnp.where(is_max, iota, _E), axis=0, keepdims=True)
        sel = iota == first
        rows.append(first)
        mask = jnp.logical_or(mask, sel)
        work = jnp.where(sel, -3.0e38, work)
    idx_ref[...] = jnp.concatenate(rows, axis=0)
    wm = jnp.where(mask, p, 0.0)
    wsum = jnp.sum(wm, axis=0, keepdims=True)
    w = wm / wsum
    pmix_ref[...] = jax.lax.dot_general(
        w, limes_ref[...], (((0,), (0,)), ((), ())),
        preferred_element_type=jnp.float32,
    )


@jax.jit
def kernel(H, LiMEs):
    B, T, D = H.shape
    N = B * T
    HsT = H.reshape(N, D)[:, :_E].T

    grid = (N // _TILE,)
    pmix, idx = pl.pallas_call(
        _mix_kernel,
        grid=grid,
        in_specs=[
            pl.BlockSpec((_E, N), lambda i: (0, 0)),
            pl.BlockSpec((_E, D), lambda i: (0, 0)),
        ],
        out_specs=[
            pl.BlockSpec((_TILE, D), lambda i: (i, 0)),
            pl.BlockSpec((_K, _TILE), lambda i: (0, i)),
        ],
        out_shape=[
            jax.ShapeDtypeStruct((N, D), jnp.float32),
            jax.ShapeDtypeStruct((_K, N), jnp.int32),
        ],
        scratch_shapes=[pltpu.SMEM((1, 1), jnp.float32)],
    )(HsT, LiMEs)

    return pmix.reshape(B, T, D), idx.T.reshape(B, T, _K)
